# Initial kernel scaffold; baseline (speedup 1.0000x reference)
import jax
import jax.numpy as jnp
from jax.experimental import pallas as pl


def kernel(pts3D, src, default_feature):
    raise NotImplementedError("write your pallas kernel here")


# speedup vs baseline: 19.3123x; 19.3123x over previous
import functools

import jax
import jax.numpy as jnp
from jax.experimental import pallas as pl
from jax.experimental.pallas import tpu as pltpu

H, W = 128, 128
K = 8
RADIUS_PX = 1.5
BIG = 1e10


def _row_kernel(pts_ref, src_ref, o_ref, *, r2, inv_r2):
    y = pl.program_id(1)
    pts = pts_ref[0]
    px = pts[0:1, :]
    py = pts[1:2, :]
    pz = pts[2:3, :]
    P = px.shape[1]

    s = float(min(H, W))
    xi = jax.lax.broadcasted_iota(jnp.int32, (W, 1), 0).astype(jnp.float32)
    xf = (2.0 * (W - 1.0 - xi) + 1.0 - W) * (1.0 / s)
    yf = (2.0 * (H - 1.0) + 1.0 - H - 2.0 * y.astype(jnp.float32)) * (1.0 / s)

    dx = xf - px
    dy = yf - py
    dist2 = dx * dx + dy * dy
    valid = (dist2 < r2) & (pz > 0.0)
    zb = jnp.broadcast_to(pz, (W, P))
    zv = jnp.where(valid, zb, BIG)
    alpha = 1.0 - jnp.sqrt(jnp.clip(dist2 * inv_r2, 0.001, 1.0))

    A = jnp.zeros((W, P), jnp.float32)
    for _ in range(K):
        m = jnp.min(zv, axis=1, keepdims=True)
        sel = (zv == m) & (m < BIG)
        A = jnp.where(sel, alpha, A)
        zv = jnp.where(sel, BIG, zv)

    o_ref[0, 0] = jnp.dot(A, src_ref[0], preferred_element_type=jnp.float32)


def _forward_tc(pts3D, srcT, *, interpret=False):
    bs, P, _ = pts3D.shape
    C = srcT.shape[2]
    radius = float(RADIUS_PX) / float(max(H, W)) * 2.0
    r2 = radius * radius
    inv_r2 = 1.0 / r2
    pts_l = jnp.transpose(pts3D, (0, 2, 1))

    out = pl.pallas_call(
        functools.partial(_row_kernel, r2=r2, inv_r2=inv_r2),
        grid=(bs, H),
        in_specs=[
            pl.BlockSpec((1, 3, P), lambda b, y: (b, 0, 0)),
            pl.BlockSpec((1, P, C), lambda b, y: (b, 0, 0)),
        ],
        out_specs=pl.BlockSpec((1, 1, W, C), lambda b, y: (b, y, 0, 0)),
        out_shape=jax.ShapeDtypeStruct((bs, H, W, C), jnp.float32),
        interpret=interpret,
    )(pts_l, srcT)
    return out


def kernel(pts3D, src, default_feature):
    del default_feature
    srcT = jnp.transpose(src, (0, 2, 1))
    out = _forward_tc(pts3D, srcT)
    return jnp.transpose(out, (0, 3, 1, 2))


# speedup vs baseline: 64.0431x; 3.3162x over previous
import functools

import jax
import jax.numpy as jnp
from jax import lax
from jax.experimental import pallas as pl
from jax.experimental.pallas import tpu as pltpu
from jax.experimental.pallas import tpu_sc as plsc

H, W = 128, 128
K = 8
RADIUS_PX = 1.5
BIG = 1e10
NBINS = 134
CW = 256
DAUG = 128


def _sc_bin_body(ypts_hbm, aug_hbm, saug_hbm, lo_hbm, hi_hbm, perm_hbm,
                 y_v, b_v, hist_v, offs_v, start_v, perm_v, lo_v, hi_v,
                 idx_v, buf_v, sem, *, P):
    c = lax.axis_index("c")
    s = lax.axis_index("s")
    lidx = lax.broadcasted_iota(jnp.int32, (16,), 0)
    lane0 = lidx == 0

    @pl.when(s == 0)
    def _phase1():
        pltpu.sync_copy(ypts_hbm.at[c], y_v)

        def zero(i, _):
            hist_v[pl.ds(i * 16, 16)] = jnp.zeros((16,), jnp.int32)
            return 0

        lax.fori_loop(0, NBINS, zero, 0)

        def binb(i, _):
            yv = y_v[pl.ds(i * 16, 16)]
            ypix = 63.5 - 64.0 * yv
            tr = ypix.astype(jnp.int32)
            fl = tr - jnp.where(ypix < tr.astype(jnp.float32), 1, 0)
            bb = jnp.clip(fl, -3, 130) + 3
            b_v[pl.ds(i * 16, 16)] = bb
            hidx = bb * 16 + lidx
            cnt = plsc.load_gather(hist_v, [hidx])
            plsc.store_scatter(hist_v, [hidx], cnt + 1)
            return 0

        lax.fori_loop(0, P // 16, binb, 0)

        def prow(r, run):
            row = hist_v[pl.ds(r * 16, 16)]
            ex = plsc.cumsum(row) - row
            offs_v[pl.ds(r * 16, 16)] = ex + run
            plsc.store_scatter(start_v, [jnp.full((16,), r, jnp.int32)],
                               run, mask=lane0)
            tot = jnp.sum(row)
            return run + jnp.full((16,), tot, jnp.int32)

        run = lax.fori_loop(0, NBINS, prow, jnp.zeros((16,), jnp.int32))
        plsc.store_scatter(start_v, [jnp.full((16,), NBINS, jnp.int32)],
                           run, mask=lane0)
        lax.fori_loop(0, NBINS, zero, 0)

        def destb(i, _):
            bb = b_v[pl.ds(i * 16, 16)]
            hidx = bb * 16 + lidx
            base = plsc.load_gather(offs_v, [hidx])
            kk = plsc.load_gather(hist_v, [hidx])
            plsc.store_scatter(hist_v, [hidx], kk + 1)
            pid = jnp.full((16,), i * 16, jnp.int32) + lidx
            plsc.store_scatter(perm_v, [base + kk], pid)
            return 0

        lax.fori_loop(0, P // 16, destb, 0)

        def rows(i, _):
            rr = jnp.full((16,), i * 16, jnp.int32) + lidx
            lo_v[pl.ds(i * 16, 16)] = plsc.load_gather(start_v, [rr + 1])
            hi_v[pl.ds(i * 16, 16)] = plsc.load_gather(start_v, [rr + 5])
            return 0

        lax.fori_loop(0, H // 16, rows, 0)
        pltpu.sync_copy(lo_v, lo_hbm.at[c])
        pltpu.sync_copy(hi_v, hi_hbm.at[c])
        pltpu.sync_copy(perm_v, perm_hbm.at[c])

    plsc.subcore_barrier()

    rows_per = P // 16
    base = s * rows_per
    pltpu.sync_copy(perm_hbm.at[c, pl.ds(base, rows_per)], idx_v)
    pltpu.async_copy(aug_hbm.at[c].at[idx_v], buf_v, sem).wait()
    pltpu.sync_copy(buf_v, saug_hbm.at[c, pl.ds(base, rows_per)])


def _sc_bin_call(ypts, aug, *, interpret=False):
    bs, P = ypts.shape
    mesh = plsc.VectorSubcoreMesh(core_axis_name="c", subcore_axis_name="s")
    return pl.kernel(
        functools.partial(_sc_bin_body, P=P),
        out_type=[
            jax.ShapeDtypeStruct((bs, P, DAUG), jnp.float32),
            jax.ShapeDtypeStruct((bs, H), jnp.int32),
            jax.ShapeDtypeStruct((bs, H), jnp.int32),
            jax.ShapeDtypeStruct((bs, P), jnp.int32),
        ],
        mesh=mesh,
        scratch_types=[
            pltpu.VMEM((P,), jnp.float32),
            pltpu.VMEM((P,), jnp.int32),
            pltpu.VMEM((NBINS * 16,), jnp.int32),
            pltpu.VMEM((NBINS * 16,), jnp.int32),
            pltpu.VMEM((136,), jnp.int32),
            pltpu.VMEM((P,), jnp.int32),
            pltpu.VMEM((H,), jnp.int32),
            pltpu.VMEM((H,), jnp.int32),
            pltpu.VMEM((P // 16,), jnp.int32),
            pltpu.VMEM((P // 16, DAUG), jnp.float32),
            pltpu.SemaphoreType.DMA,
        ],
        compiler_params=pltpu.CompilerParams(needs_layout_passes=False),
        interpret=interpret,
    )(ypts, aug)


def _tc_row_body(lo_sm, hi_sm, saug_ref, o_ref, *, r2, inv_r2, P, C):
    b = pl.program_id(0)
    y = pl.program_id(1)
    lo = lo_sm[b, y]
    hi = hi_sm[b, y]
    lo_al = (lo // 8) * 8
    n = hi - lo_al
    wstart = pl.multiple_of(jnp.minimum(lo_al, P - CW), 8)

    s = float(min(H, W))
    xi = lax.broadcasted_iota(jnp.int32, (1, W), 1).astype(jnp.float32)
    xf = (2.0 * (W - 1.0) + 1.0 - W - 2.0 * xi) * (1.0 / s)
    yf = (2.0 * (H - 1.0) + 1.0 - H - 2.0 * y.astype(jnp.float32)) * (1.0 / s)

    def compute(w):
        px = w[:, C:C + 1]
        py = w[:, C + 1:C + 2]
        pz = w[:, C + 2:C + 3]
        dx = xf - px
        dy = yf - py
        dist2 = dx * dx + dy * dy
        valid = (dist2 < r2) & (pz > 0.0)
        zv = jnp.where(valid, jnp.broadcast_to(pz, dist2.shape), BIG)
        alpha = 1.0 - jnp.sqrt(jnp.clip(dist2 * inv_r2, 0.001, 1.0))
        A = jnp.zeros_like(dist2)
        for _ in range(K):
            m = jnp.min(zv, axis=0, keepdims=True)
            sel = (zv == m) & (m < BIG)
            A = jnp.where(sel, alpha, A)
            zv = jnp.where(sel, BIG, zv)
        return lax.dot_general(A, w[:, 0:C], (((0,), (0,)), ((), ())),
                               preferred_element_type=jnp.float32)

    def fast():
        o_ref[0, 0] = compute(saug_ref[0, pl.ds(wstart, CW), :])

    def slow():
        o_ref[0, 0] = compute(saug_ref[0, :, :])

    lax.cond(n <= CW, fast, slow)


def _tc_row_call(lo, hi, saug, *, r2, inv_r2, C, interpret=False):
    bs, P, _ = saug.shape
    grid_spec = pltpu.PrefetchScalarGridSpec(
        num_scalar_prefetch=2,
        grid=(bs, H),
        in_specs=[pl.BlockSpec((1, P, DAUG), lambda b, y, *_: (b, 0, 0))],
        out_specs=pl.BlockSpec((1, 1, W, C), lambda b, y, *_: (b, y, 0, 0)),
    )
    return pl.pallas_call(
        functools.partial(_tc_row_body, r2=r2, inv_r2=inv_r2, P=P, C=C),
        grid_spec=grid_spec,
        out_shape=jax.ShapeDtypeStruct((bs, H, W, C), jnp.float32),
        interpret=interpret,
    )(lo, hi, saug)


def kernel(pts3D, src, default_feature):
    del default_feature
    bs, P, _ = pts3D.shape
    C = src.shape[1]
    radius = float(RADIUS_PX) / float(max(H, W)) * 2.0
    r2 = radius * radius
    inv_r2 = 1.0 / r2

    srcT = jnp.transpose(src, (0, 2, 1))
    pad = jnp.zeros((bs, P, DAUG - C - 3), jnp.float32)
    aug = jnp.concatenate([srcT, pts3D, pad], axis=2)
    ypts = pts3D[:, :, 1]

    saug, lo, hi, _perm = _sc_bin_call(ypts, aug)
    out = _tc_row_call(lo, hi, saug, r2=r2, inv_r2=inv_r2, C=C)
    return jnp.transpose(out, (0, 3, 1, 2))


# speedup vs baseline: 76.5695x; 1.1956x over previous
import functools

import jax
import jax.numpy as jnp
from jax import lax
from jax.experimental import pallas as pl
from jax.experimental.pallas import tpu as pltpu
from jax.experimental.pallas import tpu_sc as plsc

H, W = 128, 128
K = 8
RADIUS_PX = 1.5
BIG = 1e10
NBINS = 134
CW = 128
DAUG = 128


def _sc_bin_body(ypts_hbm, aug_hbm, saug_hbm, lo_hbm, hi_hbm, perm_hbm,
                 y_v, b_v, hist_v, offs_v, start_v, perm_v, lo_v, hi_v,
                 idx_v, buf_v, sem, *, P):
    c = lax.axis_index("c")
    s = lax.axis_index("s")
    lidx = lax.broadcasted_iota(jnp.int32, (16,), 0)
    lane0 = lidx == 0

    @pl.when(s == 0)
    def _phase1():
        pltpu.sync_copy(ypts_hbm.at[c], y_v)

        def zero(i, _):
            hist_v[pl.ds(i * 16, 16)] = jnp.zeros((16,), jnp.int32)
            return 0

        lax.fori_loop(0, NBINS, zero, 0)

        def binb(i, _):
            yv = y_v[pl.ds(i * 16, 16)]
            ypix = 63.5 - 64.0 * yv
            tr = ypix.astype(jnp.int32)
            fl = tr - jnp.where(ypix < tr.astype(jnp.float32), 1, 0)
            bb = jnp.clip(fl, -3, 130) + 3
            b_v[pl.ds(i * 16, 16)] = bb
            hidx = bb * 16 + lidx
            cnt = plsc.load_gather(hist_v, [hidx])
            plsc.store_scatter(hist_v, [hidx], cnt + 1)
            return 0

        lax.fori_loop(0, P // 16, binb, 0)

        def prow(r, run):
            row = hist_v[pl.ds(r * 16, 16)]
            ex = plsc.cumsum(row) - row
            offs_v[pl.ds(r * 16, 16)] = ex + run
            plsc.store_scatter(start_v, [jnp.full((16,), r, jnp.int32)],
                               run, mask=lane0)
            tot = jnp.sum(row)
            return run + jnp.full((16,), tot, jnp.int32)

        run = lax.fori_loop(0, NBINS, prow, jnp.zeros((16,), jnp.int32))
        plsc.store_scatter(start_v, [jnp.full((16,), NBINS, jnp.int32)],
                           run, mask=lane0)
        lax.fori_loop(0, NBINS, zero, 0)

        def destb(i, _):
            bb = b_v[pl.ds(i * 16, 16)]
            hidx = bb * 16 + lidx
            base = plsc.load_gather(offs_v, [hidx])
            kk = plsc.load_gather(hist_v, [hidx])
            plsc.store_scatter(hist_v, [hidx], kk + 1)
            pid = jnp.full((16,), i * 16, jnp.int32) + lidx
            plsc.store_scatter(perm_v, [base + kk], pid)
            return 0

        lax.fori_loop(0, P // 16, destb, 0)

        def rows(i, _):
            rr = jnp.full((16,), i * 16, jnp.int32) + lidx
            lo_v[pl.ds(i * 16, 16)] = plsc.load_gather(start_v, [rr + 1])
            hi_v[pl.ds(i * 16, 16)] = plsc.load_gather(start_v, [rr + 5])
            return 0

        lax.fori_loop(0, H // 16, rows, 0)
        pltpu.sync_copy(lo_v, lo_hbm.at[c])
        pltpu.sync_copy(hi_v, hi_hbm.at[c])
        pltpu.sync_copy(perm_v, perm_hbm.at[c])

    plsc.subcore_barrier()

    rows_per = P // 16
    base = s * rows_per
    pltpu.sync_copy(perm_hbm.at[c, pl.ds(base, rows_per)], idx_v)
    pltpu.async_copy(aug_hbm.at[c].at[idx_v], buf_v, sem).wait()
    pltpu.sync_copy(buf_v, saug_hbm.at[c, pl.ds(base, rows_per)])


def _sc_bin_call(ypts, aug, *, interpret=False):
    bs, P = ypts.shape
    mesh = plsc.VectorSubcoreMesh(core_axis_name="c", subcore_axis_name="s")
    return pl.kernel(
        functools.partial(_sc_bin_body, P=P),
        out_type=[
            jax.ShapeDtypeStruct((bs, P, DAUG), jnp.float32),
            jax.ShapeDtypeStruct((bs, H), jnp.int32),
            jax.ShapeDtypeStruct((bs, H), jnp.int32),
            jax.ShapeDtypeStruct((bs, P), jnp.int32),
        ],
        mesh=mesh,
        scratch_types=[
            pltpu.VMEM((P,), jnp.float32),
            pltpu.VMEM((P,), jnp.int32),
            pltpu.VMEM((NBINS * 16,), jnp.int32),
            pltpu.VMEM((NBINS * 16,), jnp.int32),
            pltpu.VMEM((136,), jnp.int32),
            pltpu.VMEM((P,), jnp.int32),
            pltpu.VMEM((H,), jnp.int32),
            pltpu.VMEM((H,), jnp.int32),
            pltpu.VMEM((P // 16,), jnp.int32),
            pltpu.VMEM((P // 16, DAUG), jnp.float32),
            pltpu.SemaphoreType.DMA,
        ],
        compiler_params=pltpu.CompilerParams(needs_layout_passes=False),
        interpret=interpret,
    )(ypts, aug)


def _tc_row_body(lo_sm, hi_sm, saug_ref, o_ref, *, r2, inv_r2, P, C):
    b = pl.program_id(0)
    y = pl.program_id(1)
    lo = lo_sm[b, y]
    hi = hi_sm[b, y]
    lo_al = (lo // 8) * 8
    n = hi - lo_al

    s = float(min(H, W))
    xi = lax.broadcasted_iota(jnp.int32, (1, W), 1).astype(jnp.float32)
    xf = (2.0 * (W - 1.0) + 1.0 - W - 2.0 * xi) * (1.0 / s)
    yf = (2.0 * (H - 1.0) + 1.0 - H - 2.0 * y.astype(jnp.float32)) * (1.0 / s)

    def compute(w):
        px = w[:, C:C + 1]
        py = w[:, C + 1:C + 2]
        pz = w[:, C + 2:C + 3]
        dx = xf - px
        dy = yf - py
        dist2 = dx * dx + dy * dy
        valid = (dist2 < r2) & (pz > 0.0)
        alpha = 1.0 - jnp.sqrt(jnp.clip(dist2 * inv_r2, 0.001, 1.0))
        cnt = jnp.sum(jnp.where(valid, 1, 0), axis=0)
        maxc = jnp.max(cnt)

        def direct():
            return jnp.where(valid, alpha, 0.0)

        def extract():
            zv = jnp.where(valid, jnp.broadcast_to(pz, dist2.shape), BIG)
            A = jnp.zeros_like(dist2)
            for _ in range(K):
                m = jnp.min(zv, axis=0, keepdims=True)
                sel = (zv == m) & (m < BIG)
                A = jnp.where(sel, alpha, A)
                zv = jnp.where(sel, BIG, zv)
            return A

        A = lax.cond(maxc <= K, direct, extract)
        return lax.dot_general(A, w[:, 0:C], (((0,), (0,)), ((), ())),
                               preferred_element_type=jnp.float32)

    def window(width):
        def go():
            ws = pl.multiple_of(jnp.minimum(lo_al, P - width), 8)
            o_ref[0, 0] = compute(saug_ref[0, pl.ds(ws, width), :])
        return go

    def full():
        o_ref[0, 0] = compute(saug_ref[0, :, :])

    lax.cond(n <= CW, window(CW),
             lambda: lax.cond(n <= 4 * CW, window(4 * CW), full))


def _tc_row_call(lo, hi, saug, *, r2, inv_r2, C, interpret=False):
    bs, P, _ = saug.shape
    grid_spec = pltpu.PrefetchScalarGridSpec(
        num_scalar_prefetch=2,
        grid=(bs, H),
        in_specs=[pl.BlockSpec((1, P, DAUG), lambda b, y, *_: (b, 0, 0))],
        out_specs=pl.BlockSpec((1, 1, W, C), lambda b, y, *_: (b, y, 0, 0)),
    )
    return pl.pallas_call(
        functools.partial(_tc_row_body, r2=r2, inv_r2=inv_r2, P=P, C=C),
        grid_spec=grid_spec,
        out_shape=jax.ShapeDtypeStruct((bs, H, W, C), jnp.float32),
        interpret=interpret,
    )(lo, hi, saug)


def kernel(pts3D, src, default_feature):
    del default_feature
    bs, P, _ = pts3D.shape
    C = src.shape[1]
    radius = float(RADIUS_PX) / float(max(H, W)) * 2.0
    r2 = radius * radius
    inv_r2 = 1.0 / r2

    srcT = jnp.transpose(src, (0, 2, 1))
    pad = jnp.zeros((bs, P, DAUG - C - 3), jnp.float32)
    aug = jnp.concatenate([srcT, pts3D, pad], axis=2)
    ypts = pts3D[:, :, 1]

    saug, lo, hi, _perm = _sc_bin_call(ypts, aug)
    out = _tc_row_call(lo, hi, saug, r2=r2, inv_r2=inv_r2, C=C)
    return jnp.transpose(out, (0, 3, 1, 2))


# speedup vs baseline: 133.5873x; 1.7447x over previous
import functools

import jax
import jax.numpy as jnp
from jax import lax
from jax.experimental import pallas as pl
from jax.experimental.pallas import tpu as pltpu
from jax.experimental.pallas import tpu_sc as plsc

H, W = 128, 128
K = 8
RADIUS_PX = 1.5
BIG = 1e10
NBINS = 134
CW = 256
RG = 4
DAUG = 128


def _sc_bin_body(ypts_hbm, aug_hbm, saug_hbm, lo_hbm, hi_hbm, perm_hbm,
                 y_v, b_v, hist_v, offs_v, start_v, perm_v, lo_v, hi_v,
                 idx_v, buf_v, sem, *, P):
    c = lax.axis_index("c")
    s = lax.axis_index("s")
    lidx = lax.broadcasted_iota(jnp.int32, (16,), 0)
    lane0 = lidx == 0

    @pl.when(s == 0)
    def _phase1():
        pltpu.sync_copy(ypts_hbm.at[c], y_v)

        def zero(i, _):
            hist_v[pl.ds(i * 16, 16)] = jnp.zeros((16,), jnp.int32)
            return 0

        lax.fori_loop(0, NBINS, zero, 0)

        def binb(i, _):
            yv = y_v[pl.ds(i * 16, 16)]
            ypix = 63.5 - 64.0 * yv
            tr = ypix.astype(jnp.int32)
            fl = tr - jnp.where(ypix < tr.astype(jnp.float32), 1, 0)
            bb = jnp.clip(fl, -3, 130) + 3
            b_v[pl.ds(i * 16, 16)] = bb
            hidx = bb * 16 + lidx
            cnt = plsc.load_gather(hist_v, [hidx])
            plsc.store_scatter(hist_v, [hidx], cnt + 1)
            return 0

        lax.fori_loop(0, P // 16, binb, 0)

        def prow(r, run):
            row = hist_v[pl.ds(r * 16, 16)]
            ex = plsc.cumsum(row) - row
            offs_v[pl.ds(r * 16, 16)] = ex + run
            plsc.store_scatter(start_v, [jnp.full((16,), r, jnp.int32)],
                               run, mask=lane0)
            tot = jnp.sum(row)
            return run + jnp.full((16,), tot, jnp.int32)

        run = lax.fori_loop(0, NBINS, prow, jnp.zeros((16,), jnp.int32))
        plsc.store_scatter(start_v, [jnp.full((16,), NBINS, jnp.int32)],
                           run, mask=lane0)
        lax.fori_loop(0, NBINS, zero, 0)

        def destb(i, _):
            bb = b_v[pl.ds(i * 16, 16)]
            hidx = bb * 16 + lidx
            base = plsc.load_gather(offs_v, [hidx])
            kk = plsc.load_gather(hist_v, [hidx])
            plsc.store_scatter(hist_v, [hidx], kk + 1)
            pid = jnp.full((16,), i * 16, jnp.int32) + lidx
            plsc.store_scatter(perm_v, [base + kk], pid)
            return 0

        lax.fori_loop(0, P // 16, destb, 0)

        def rows(i, _):
            rr = jnp.full((16,), i * 16, jnp.int32) + lidx
            lo_v[pl.ds(i * 16, 16)] = plsc.load_gather(start_v, [rr + 1])
            hi_v[pl.ds(i * 16, 16)] = plsc.load_gather(start_v, [rr + 5])
            return 0

        lax.fori_loop(0, H // 16, rows, 0)
        pltpu.sync_copy(lo_v, lo_hbm.at[c])
        pltpu.sync_copy(hi_v, hi_hbm.at[c])
        pltpu.sync_copy(perm_v, perm_hbm.at[c])

    plsc.subcore_barrier()

    rows_per = P // 16
    base = s * rows_per
    pltpu.sync_copy(perm_hbm.at[c, pl.ds(base, rows_per)], idx_v)
    pltpu.async_copy(aug_hbm.at[c].at[idx_v], buf_v, sem).wait()
    pltpu.sync_copy(buf_v, saug_hbm.at[c, pl.ds(base, rows_per)])


def _sc_bin_call(ypts, aug, *, interpret=False):
    bs, P = ypts.shape
    mesh = plsc.VectorSubcoreMesh(core_axis_name="c", subcore_axis_name="s")
    return pl.kernel(
        functools.partial(_sc_bin_body, P=P),
        out_type=[
            jax.ShapeDtypeStruct((bs, P, DAUG), jnp.float32),
            jax.ShapeDtypeStruct((bs, H), jnp.int32),
            jax.ShapeDtypeStruct((bs, H), jnp.int32),
            jax.ShapeDtypeStruct((bs, P), jnp.int32),
        ],
        mesh=mesh,
        scratch_types=[
            pltpu.VMEM((P,), jnp.float32),
            pltpu.VMEM((P,), jnp.int32),
            pltpu.VMEM((NBINS * 16,), jnp.int32),
            pltpu.VMEM((NBINS * 16,), jnp.int32),
            pltpu.VMEM((136,), jnp.int32),
            pltpu.VMEM((P,), jnp.int32),
            pltpu.VMEM((H,), jnp.int32),
            pltpu.VMEM((H,), jnp.int32),
            pltpu.VMEM((P // 16,), jnp.int32),
            pltpu.VMEM((P // 16, DAUG), jnp.float32),
            pltpu.SemaphoreType.DMA,
        ],
        compiler_params=pltpu.CompilerParams(needs_layout_passes=False),
        interpret=interpret,
    )(ypts, aug)


def _tc_row_body(lo_sm, hi_sm, saug_ref, o_ref, *, r2, inv_r2, P, C):
    b = pl.program_id(0)
    g = pl.program_id(1)
    y0 = g * RG
    lo = lo_sm[b, y0]
    hi = hi_sm[b, y0 + (RG - 1)]
    lo_al = (lo // 8) * 8
    n = hi - lo_al

    s = float(min(H, W))
    L = RG * W
    li = lax.broadcasted_iota(jnp.int32, (1, L), 1)
    xi = (li % W).astype(jnp.float32)
    yi = (li // W + y0).astype(jnp.float32)
    xf = (2.0 * (W - 1.0) + 1.0 - W - 2.0 * xi) * (1.0 / s)
    yf = (2.0 * (H - 1.0) + 1.0 - H - 2.0 * yi) * (1.0 / s)

    def compute(w):
        px = w[:, C:C + 1]
        py = w[:, C + 1:C + 2]
        pz = w[:, C + 2:C + 3]
        dx = xf - px
        dy = yf - py
        dist2 = dx * dx + dy * dy
        valid = (dist2 < r2) & (pz > 0.0)
        alpha = 1.0 - jnp.sqrt(jnp.clip(dist2 * inv_r2, 0.001, 1.0))
        cnt = jnp.sum(jnp.where(valid, 1, 0), axis=0)
        maxc = jnp.max(cnt)

        def direct():
            return jnp.where(valid, alpha, 0.0)

        def extract():
            zv = jnp.where(valid, jnp.broadcast_to(pz, dist2.shape), BIG)
            A = jnp.zeros_like(dist2)
            for _ in range(K):
                m = jnp.min(zv, axis=0, keepdims=True)
                sel = (zv == m) & (m < BIG)
                A = jnp.where(sel, alpha, A)
                zv = jnp.where(sel, BIG, zv)
            return A

        A = lax.cond(maxc <= K, direct, extract)
        out = lax.dot_general(A, w[:, 0:C], (((0,), (0,)), ((), ())),
                              preferred_element_type=jnp.float32)
        return out.reshape(RG, W, C)

    def window(width):
        def go():
            ws = pl.multiple_of(jnp.minimum(lo_al, P - width), 8)
            o_ref[0] = compute(saug_ref[0, pl.ds(ws, width), :])
        return go

    def full():
        o_ref[0] = compute(saug_ref[0, :, :])

    lax.cond(n <= CW, window(CW),
             lambda: lax.cond(n <= 4 * CW, window(4 * CW), full))


def _tc_row_call(lo, hi, saug, *, r2, inv_r2, C, interpret=False):
    bs, P, _ = saug.shape
    grid_spec = pltpu.PrefetchScalarGridSpec(
        num_scalar_prefetch=2,
        grid=(bs, H // RG),
        in_specs=[pl.BlockSpec((1, P, DAUG), lambda b, g, *_: (b, 0, 0))],
        out_specs=pl.BlockSpec((1, RG, W, C), lambda b, g, *_: (b, g, 0, 0)),
    )
    return pl.pallas_call(
        functools.partial(_tc_row_body, r2=r2, inv_r2=inv_r2, P=P, C=C),
        grid_spec=grid_spec,
        out_shape=jax.ShapeDtypeStruct((bs, H, W, C), jnp.float32),
        interpret=interpret,
    )(lo, hi, saug)


def kernel(pts3D, src, default_feature):
    del default_feature
    bs, P, _ = pts3D.shape
    C = src.shape[1]
    radius = float(RADIUS_PX) / float(max(H, W)) * 2.0
    r2 = radius * radius
    inv_r2 = 1.0 / r2

    srcT = jnp.transpose(src, (0, 2, 1))
    pad = jnp.zeros((bs, P, DAUG - C - 3), jnp.float32)
    aug = jnp.concatenate([srcT, pts3D, pad], axis=2)
    ypts = pts3D[:, :, 1]

    saug, lo, hi, _perm = _sc_bin_call(ypts, aug)
    out = _tc_row_call(lo, hi, saug, r2=r2, inv_r2=inv_r2, C=C)
    return jnp.transpose(out, (0, 3, 1, 2))
